# Initial kernel scaffold; baseline (speedup 1.0000x reference)
import jax
import jax.numpy as jnp
from jax.experimental import pallas as pl


def kernel(x, adj, weight, bias):
    raise NotImplementedError("write your pallas kernel here")


# speedup vs baseline: 1.0058x; 1.0058x over previous
import functools

import jax
import jax.numpy as jnp
from jax.experimental import pallas as pl


def _row_block(n: int, cap: int = 256) -> int:
    best = 8
    for b in range(8, cap + 1, 8):
        if n % b == 0:
            best = b
    return best


def _spmm_body(adj_ref, x_ref, o_ref):
    o_ref[...] = jnp.dot(adj_ref[...], x_ref[...],
                         preferred_element_type=jnp.float32)


def _fused_body(adj_ref, tx1_full_ref, x_ref, tx1_ref, w_ref, b_ref, o_ref):
    y = jnp.dot(adj_ref[...], tx1_full_ref[...],
                preferred_element_type=jnp.float32)
    xb = x_ref[...]
    acc = jnp.dot(xb, w_ref[0], preferred_element_type=jnp.float32)
    acc = acc + jnp.dot(tx1_ref[...], w_ref[1],
                        preferred_element_type=jnp.float32)
    tx2 = 2.0 * y - xb
    acc = acc + jnp.dot(tx2, w_ref[2], preferred_element_type=jnp.float32)
    o_ref[...] = acc + b_ref[...]


@functools.partial(jax.jit, static_argnames=())
def kernel(x, adj, weight, bias):
    n, d = x.shape
    bm = _row_block(n)
    grid = (n // bm,)
    bias2 = bias.reshape(1, d)

    tx1 = pl.pallas_call(
        _spmm_body,
        grid=grid,
        in_specs=[
            pl.BlockSpec((bm, n), lambda i: (i, 0)),
            pl.BlockSpec((n, d), lambda i: (0, 0)),
        ],
        out_specs=pl.BlockSpec((bm, d), lambda i: (i, 0)),
        out_shape=jax.ShapeDtypeStruct((n, d), jnp.float32),
    )(adj, x)

    out = pl.pallas_call(
        _fused_body,
        grid=grid,
        in_specs=[
            pl.BlockSpec((bm, n), lambda i: (i, 0)),
            pl.BlockSpec((n, d), lambda i: (0, 0)),
            pl.BlockSpec((bm, d), lambda i: (i, 0)),
            pl.BlockSpec((bm, d), lambda i: (i, 0)),
            pl.BlockSpec(weight.shape, lambda i: (0, 0, 0)),
            pl.BlockSpec((1, d), lambda i: (0, 0)),
        ],
        out_specs=pl.BlockSpec((bm, d), lambda i: (i, 0)),
        out_shape=jax.ShapeDtypeStruct((n, d), jnp.float32),
    )(adj, tx1, x, tx1, weight, bias2)
    return out


# speedup vs baseline: 1.0440x; 1.0379x over previous
import functools

import jax
import jax.numpy as jnp
from jax.experimental import pallas as pl
from jax.experimental.pallas import tpu as pltpu


def _row_block(n: int, cap: int = 256) -> int:
    best = 8
    for b in range(8, cap + 1, 8):
        if n % b == 0:
            best = b
    return best


def _cheb_body(adj_ref, x_ref, w_ref, b_ref, o_ref, tx1_ref):
    p = pl.program_id(0)
    i = pl.program_id(1)
    bm = adj_ref.shape[0]

    @pl.when(p == 0)
    def _phase0():
        tx1_ref[pl.ds(i * bm, bm), :] = jnp.dot(
            adj_ref[...], x_ref[...], preferred_element_type=jnp.float32)

    @pl.when(p == 1)
    def _phase1():
        y = jnp.dot(adj_ref[...], tx1_ref[...],
                    preferred_element_type=jnp.float32)
        xb = x_ref[pl.ds(i * bm, bm), :]
        acc = jnp.dot(xb, w_ref[0], preferred_element_type=jnp.float32)
        acc = acc + jnp.dot(tx1_ref[pl.ds(i * bm, bm), :], w_ref[1],
                            preferred_element_type=jnp.float32)
        acc = acc + jnp.dot(2.0 * y - xb, w_ref[2],
                            preferred_element_type=jnp.float32)
        o_ref[pl.ds(i * bm, bm), :] = acc + b_ref[...]


def kernel(x, adj, weight, bias):
    n, d = x.shape
    bm = _row_block(n)
    bias2 = bias.reshape(1, d)

    out = pl.pallas_call(
        _cheb_body,
        grid=(2, n // bm),
        in_specs=[
            pl.BlockSpec((bm, n), lambda p, i: (i, 0)),
            pl.BlockSpec((n, d), lambda p, i: (0, 0)),
            pl.BlockSpec(weight.shape, lambda p, i: (0, 0, 0)),
            pl.BlockSpec((1, d), lambda p, i: (0, 0)),
        ],
        out_specs=pl.BlockSpec((n, d), lambda p, i: (0, 0)),
        out_shape=jax.ShapeDtypeStruct((n, d), jnp.float32),
        scratch_shapes=[pltpu.VMEM((n, d), jnp.float32)],
    )(adj, x, weight, bias2)
    return out


# speedup vs baseline: 1.0669x; 1.0220x over previous
import functools

import jax
import jax.numpy as jnp
from jax.experimental import pallas as pl
from jax.experimental.pallas import tpu as pltpu


def _row_block(n: int, cap: int = 400) -> int:
    best = 8
    for b in range(8, cap + 1, 8):
        if n % b == 0:
            best = b
    return best


def _cheb_body(adj_ref, x_ref, w_ref, b_ref, o_ref, tx1_ref):
    p = pl.program_id(0)
    i = pl.program_id(1)
    bm = adj_ref.shape[0]

    @pl.when(p == 0)
    def _phase0():
        tx1_ref[pl.ds(i * bm, bm), :] = jnp.dot(
            adj_ref[...], x_ref[...], preferred_element_type=jnp.float32)

    @pl.when(p == 1)
    def _phase1():
        y = jnp.dot(adj_ref[...], tx1_ref[...],
                    preferred_element_type=jnp.float32)
        xb = x_ref[pl.ds(i * bm, bm), :]
        acc = jnp.dot(xb, w_ref[0], preferred_element_type=jnp.float32)
        acc = acc + jnp.dot(tx1_ref[pl.ds(i * bm, bm), :], w_ref[1],
                            preferred_element_type=jnp.float32)
        acc = acc + jnp.dot(2.0 * y - xb, w_ref[2],
                            preferred_element_type=jnp.float32)
        o_ref[pl.ds(i * bm, bm), :] = acc + b_ref[...]


def kernel(x, adj, weight, bias):
    n, d = x.shape
    bm = _row_block(n)
    bias2 = bias.reshape(1, d)

    out = pl.pallas_call(
        _cheb_body,
        grid=(2, n // bm),
        in_specs=[
            pl.BlockSpec((bm, n), lambda p, i: (i, 0)),
            pl.BlockSpec((n, d), lambda p, i: (0, 0)),
            pl.BlockSpec(weight.shape, lambda p, i: (0, 0, 0)),
            pl.BlockSpec((1, d), lambda p, i: (0, 0)),
        ],
        out_specs=pl.BlockSpec((n, d), lambda p, i: (0, 0)),
        out_shape=jax.ShapeDtypeStruct((n, d), jnp.float32),
        scratch_shapes=[pltpu.VMEM((n, d), jnp.float32)],
    )(adj, x, weight, bias2)
    return out
